# baseline (speedup 1.0000x reference)
import jax
import jax.numpy as jnp
from jax import lax
from jax.experimental import pallas as pl
from jax.experimental.pallas import tpu as pltpu
from jax.experimental.pallas import tpu_sc as plsc

_L = 16
_NC = 2
_NS = 16
_NW = _NC * _NS
_NPAD = 10240
_UNROLL = 8


def _sc_hist_body(idx0_hbm, idx1_hbm, out_hbm, idx0_v, idx1_v, h0_v, h1_v,
                  sem0, sem1, osem):
    wid = lax.axis_index("s") * _NC + lax.axis_index("c")
    per_w = idx0_hbm.shape[0] // _NW
    base = wid * per_w

    cp0 = pltpu.async_copy(idx0_hbm.at[pl.ds(base, per_w)], idx0_v, sem0)
    cp1 = pltpu.async_copy(idx1_hbm.at[pl.ds(base, per_w)], idx1_v, sem1)

    z = jnp.zeros((_L,), jnp.float32)

    def zero_body(i, c):
        for u in range(_UNROLL):
            h0_v[pl.ds((i * _UNROLL + u) * _L, _L)] = z
            h1_v[pl.ds((i * _UNROLL + u) * _L, _L)] = z
        return c

    lax.fori_loop(0, _NPAD // (_L * _UNROLL), zero_body, 0)

    ones = jnp.ones((_L,), jnp.float32)
    n_full = per_w // _L
    rem = per_w - n_full * _L
    n_unrolled = n_full // _UNROLL

    def scatter_all(idx_v, h_v):
        def body(i, c):
            for u in range(_UNROLL):
                idx = idx_v[pl.ds((i * _UNROLL + u) * _L, _L)]
                plsc.addupdate_scatter(h_v, [idx], ones)
            return c

        lax.fori_loop(0, n_unrolled, body, 0)
        for j in range(n_unrolled * _UNROLL, n_full):
            idx = idx_v[pl.ds(j * _L, _L)]
            plsc.addupdate_scatter(h_v, [idx], ones)
        if rem:
            idx = idx_v[pl.ds(per_w - _L, _L)]
            mask = lax.iota(jnp.int32, _L) >= (_L - rem)
            plsc.addupdate_scatter(h_v, [idx], ones, mask=mask)

    cp0.wait()
    scatter_all(idx0_v, h0_v)
    ocp = pltpu.async_copy(h0_v, out_hbm.at[0, wid], osem)
    cp1.wait()
    scatter_all(idx1_v, h1_v)
    ocp.wait()
    pltpu.sync_copy(h1_v, out_hbm.at[1, wid])


def _sc_histogram(idx0, idx1):
    per_w = idx0.shape[0] // _NW
    mesh = plsc.VectorSubcoreMesh(core_axis_name="c", subcore_axis_name="s")
    return pl.kernel(
        _sc_hist_body,
        mesh=mesh,
        out_type=jax.ShapeDtypeStruct((2, _NW, _NPAD), jnp.float32),
        scratch_types=[
            pltpu.VMEM((per_w,), jnp.int32),
            pltpu.VMEM((per_w,), jnp.int32),
            pltpu.VMEM((_NPAD,), jnp.float32),
            pltpu.VMEM((_NPAD,), jnp.float32),
            pltpu.SemaphoreType.DMA,
            pltpu.SemaphoreType.DMA,
            pltpu.SemaphoreType.DMA,
        ],
        compiler_params=pltpu.CompilerParams(needs_layout_passes=False),
    )(idx0, idx1)


def _tc_body(cnt_ref, emb_ref, w0_ref, w1_ref, wu_ref, b0_ref, b1_ref,
             bu_ref, out_ref):
    f32 = jnp.float32
    e = emb_ref[...]
    d = e.shape[1]
    m0 = e + jnp.dot(e, w0_ref[...], preferred_element_type=f32) + b0_ref[...]
    m1 = e + jnp.dot(e, w1_ref[...], preferred_element_type=f32) + b1_ref[...]
    cnt = cnt_ref[...]
    c0 = jnp.sum(cnt[:_NW], axis=0)[:, None]
    c1 = jnp.sum(cnt[_NW:], axis=0)[:, None]
    agg = c0 * m0 + c1 * m1
    h = (jnp.dot(e, wu_ref[:d], preferred_element_type=f32)
         + jnp.dot(agg, wu_ref[d:], preferred_element_type=f32)
         + bu_ref[...])
    out_ref[...] = jnp.maximum(h, 0.0)


def kernel(node_embeddings, rel0_indices, rel1_indices,
           W_msg_0, b_msg_0, W_msg_1, b_msg_1, W_upd, b_upd):
    n, d = node_embeddings.shape
    idx0 = rel0_indices.astype(jnp.int32)
    idx1 = rel1_indices.astype(jnp.int32)

    counts = _sc_histogram(idx0, idx1).reshape(2 * _NW, _NPAD)

    blk = 2048
    grid = (n + blk - 1) // blk
    return pl.pallas_call(
        _tc_body,
        grid=(grid,),
        in_specs=[
            pl.BlockSpec((2 * _NW, blk), lambda i: (0, i)),
            pl.BlockSpec((blk, d), lambda i: (i, 0)),
            pl.BlockSpec((d, d), lambda i: (0, 0)),
            pl.BlockSpec((d, d), lambda i: (0, 0)),
            pl.BlockSpec((2 * d, d), lambda i: (0, 0)),
            pl.BlockSpec((1, d), lambda i: (0, 0)),
            pl.BlockSpec((1, d), lambda i: (0, 0)),
            pl.BlockSpec((1, d), lambda i: (0, 0)),
        ],
        out_specs=pl.BlockSpec((blk, d), lambda i: (i, 0)),
        out_shape=jax.ShapeDtypeStruct((n, d), jnp.float32),
    )(counts, node_embeddings, W_msg_0, W_msg_1, W_upd,
      b_msg_0.reshape(1, d), b_msg_1.reshape(1, d), b_upd.reshape(1, d))


# speedup vs baseline: 1.0119x; 1.0119x over previous
import jax
import jax.numpy as jnp
from jax import lax
from jax.experimental import pallas as pl
from jax.experimental.pallas import tpu as pltpu
from jax.experimental.pallas import tpu_sc as plsc

_L = 16
_NC = 2
_NS = 16
_NW = _NC * _NS
_NPAD = 10240
_UNROLL = 8


def _sc_hist_body(idx0_hbm, idx1_hbm, out_hbm, idx0_v, idx1_v, h0_v, h1_v,
                  sem0, sem1, osem):
    wid = lax.axis_index("s") * _NC + lax.axis_index("c")
    per_w = idx0_hbm.shape[0] // _NW
    base = wid * per_w

    cp0 = pltpu.async_copy(idx0_hbm.at[pl.ds(base, per_w)], idx0_v, sem0)
    cp1 = pltpu.async_copy(idx1_hbm.at[pl.ds(base, per_w)], idx1_v, sem1)

    z = jnp.zeros((_L,), jnp.float32)

    def zero_body(i, c):
        for u in range(_UNROLL):
            h0_v[pl.ds((i * _UNROLL + u) * _L, _L)] = z
            h1_v[pl.ds((i * _UNROLL + u) * _L, _L)] = z
        return c

    lax.fori_loop(0, _NPAD // (_L * _UNROLL), zero_body, 0)

    ones = jnp.ones((_L,), jnp.float32)
    n_full = per_w // _L
    rem = per_w - n_full * _L
    n_unrolled = n_full // _UNROLL

    cp0.wait()
    cp1.wait()

    def body(i, c):
        for u in range(_UNROLL):
            off = (i * _UNROLL + u) * _L
            plsc.addupdate_scatter(h0_v, [idx0_v[pl.ds(off, _L)]], ones)
            plsc.addupdate_scatter(h1_v, [idx1_v[pl.ds(off, _L)]], ones)
        return c

    lax.fori_loop(0, n_unrolled, body, 0)
    for j in range(n_unrolled * _UNROLL, n_full):
        plsc.addupdate_scatter(h0_v, [idx0_v[pl.ds(j * _L, _L)]], ones)
        plsc.addupdate_scatter(h1_v, [idx1_v[pl.ds(j * _L, _L)]], ones)
    if rem:
        mask = lax.iota(jnp.int32, _L) >= (_L - rem)
        plsc.addupdate_scatter(h0_v, [idx0_v[pl.ds(per_w - _L, _L)]], ones, mask=mask)
        plsc.addupdate_scatter(h1_v, [idx1_v[pl.ds(per_w - _L, _L)]], ones, mask=mask)
    ocp = pltpu.async_copy(h0_v, out_hbm.at[0, wid], osem)
    pltpu.sync_copy(h1_v, out_hbm.at[1, wid])
    ocp.wait()


def _sc_histogram(idx0, idx1):
    per_w = idx0.shape[0] // _NW
    mesh = plsc.VectorSubcoreMesh(core_axis_name="c", subcore_axis_name="s")
    return pl.kernel(
        _sc_hist_body,
        mesh=mesh,
        out_type=jax.ShapeDtypeStruct((2, _NW, _NPAD), jnp.float32),
        scratch_types=[
            pltpu.VMEM((per_w,), jnp.int32),
            pltpu.VMEM((per_w,), jnp.int32),
            pltpu.VMEM((_NPAD,), jnp.float32),
            pltpu.VMEM((_NPAD,), jnp.float32),
            pltpu.SemaphoreType.DMA,
            pltpu.SemaphoreType.DMA,
            pltpu.SemaphoreType.DMA,
        ],
        compiler_params=pltpu.CompilerParams(needs_layout_passes=False),
    )(idx0, idx1)


def _tc_body(cnt_ref, emb_ref, w0_ref, w1_ref, wu_ref, b0_ref, b1_ref,
             bu_ref, out_ref):
    f32 = jnp.float32
    e = emb_ref[...]
    d = e.shape[1]
    m0 = e + jnp.dot(e, w0_ref[...], preferred_element_type=f32) + b0_ref[...]
    m1 = e + jnp.dot(e, w1_ref[...], preferred_element_type=f32) + b1_ref[...]
    cnt = cnt_ref[...]
    c0 = jnp.sum(cnt[:_NW], axis=0)[:, None]
    c1 = jnp.sum(cnt[_NW:], axis=0)[:, None]
    agg = c0 * m0 + c1 * m1
    h = (jnp.dot(e, wu_ref[:d], preferred_element_type=f32)
         + jnp.dot(agg, wu_ref[d:], preferred_element_type=f32)
         + bu_ref[...])
    out_ref[...] = jnp.maximum(h, 0.0)


def kernel(node_embeddings, rel0_indices, rel1_indices,
           W_msg_0, b_msg_0, W_msg_1, b_msg_1, W_upd, b_upd):
    n, d = node_embeddings.shape
    idx0 = rel0_indices.astype(jnp.int32)
    idx1 = rel1_indices.astype(jnp.int32)

    counts = _sc_histogram(idx0, idx1).reshape(2 * _NW, _NPAD)

    blk = 2560
    grid = (n + blk - 1) // blk
    return pl.pallas_call(
        _tc_body,
        grid=(grid,),
        in_specs=[
            pl.BlockSpec((2 * _NW, blk), lambda i: (0, i)),
            pl.BlockSpec((blk, d), lambda i: (i, 0)),
            pl.BlockSpec((d, d), lambda i: (0, 0)),
            pl.BlockSpec((d, d), lambda i: (0, 0)),
            pl.BlockSpec((2 * d, d), lambda i: (0, 0)),
            pl.BlockSpec((1, d), lambda i: (0, 0)),
            pl.BlockSpec((1, d), lambda i: (0, 0)),
            pl.BlockSpec((1, d), lambda i: (0, 0)),
        ],
        out_specs=pl.BlockSpec((blk, d), lambda i: (i, 0)),
        out_shape=jax.ShapeDtypeStruct((n, d), jnp.float32),
    )(counts, node_embeddings, W_msg_0, W_msg_1, W_upd,
      b_msg_0.reshape(1, d), b_msg_1.reshape(1, d), b_upd.reshape(1, d))


# speedup vs baseline: 1.0547x; 1.0423x over previous
import jax
import jax.numpy as jnp
from jax import lax
from jax.experimental import pallas as pl
from jax.experimental.pallas import tpu as pltpu
from jax.experimental.pallas import tpu_sc as plsc

_L = 16
_NC = 2
_NS = 16
_NW = _NC * _NS
_NPAD = 10240
_UNROLL = 8


def _sc_hist_body(idx0_hbm, idx1_hbm, out_hbm, idx0_v, idx1_v, h0_v, h1_v,
                  sem0, sem1, osem):
    wid = lax.axis_index("s") * _NC + lax.axis_index("c")
    per_w = idx0_hbm.shape[0] // _NW
    base = wid * per_w

    cp0 = pltpu.async_copy(idx0_hbm.at[pl.ds(base, per_w)], idx0_v, sem0)
    cp1 = pltpu.async_copy(idx1_hbm.at[pl.ds(base, per_w)], idx1_v, sem1)

    z = jnp.zeros((_L,), jnp.float32)

    def zero_body(i, c):
        for u in range(_UNROLL):
            h0_v[pl.ds((i * _UNROLL + u) * _L, _L)] = z
            h1_v[pl.ds((i * _UNROLL + u) * _L, _L)] = z
        return c

    lax.fori_loop(0, _NPAD // (_L * _UNROLL), zero_body, 0)

    ones = jnp.ones((_L,), jnp.float32)
    n_full = per_w // _L
    rem = per_w - n_full * _L
    n_unrolled = n_full // _UNROLL

    def scatter_all(idx_v, h_v):
        def body(i, c):
            for u in range(_UNROLL):
                idx = idx_v[pl.ds((i * _UNROLL + u) * _L, _L)]
                plsc.addupdate_scatter(h_v, [idx], ones)
            return c

        lax.fori_loop(0, n_unrolled, body, 0)
        for j in range(n_unrolled * _UNROLL, n_full):
            idx = idx_v[pl.ds(j * _L, _L)]
            plsc.addupdate_scatter(h_v, [idx], ones)
        if rem:
            idx = idx_v[pl.ds(per_w - _L, _L)]
            mask = lax.iota(jnp.int32, _L) >= (_L - rem)
            plsc.addupdate_scatter(h_v, [idx], ones, mask=mask)

    cp0.wait()
    scatter_all(idx0_v, h0_v)
    ocp = pltpu.async_copy(h0_v, out_hbm.at[0, wid], osem)
    cp1.wait()
    scatter_all(idx1_v, h1_v)
    ocp.wait()
    pltpu.sync_copy(h1_v, out_hbm.at[1, wid])


def _sc_histogram(idx0, idx1):
    per_w = idx0.shape[0] // _NW
    mesh = plsc.VectorSubcoreMesh(core_axis_name="c", subcore_axis_name="s")
    return pl.kernel(
        _sc_hist_body,
        mesh=mesh,
        out_type=jax.ShapeDtypeStruct((2, _NW, _NPAD), jnp.float32),
        scratch_types=[
            pltpu.VMEM((per_w,), jnp.int32),
            pltpu.VMEM((per_w,), jnp.int32),
            pltpu.VMEM((_NPAD,), jnp.float32),
            pltpu.VMEM((_NPAD,), jnp.float32),
            pltpu.SemaphoreType.DMA,
            pltpu.SemaphoreType.DMA,
            pltpu.SemaphoreType.DMA,
        ],
        compiler_params=pltpu.CompilerParams(needs_layout_passes=False),
    )(idx0, idx1)


def _tc_body(cnt_ref, emb_ref, w0_ref, w1_ref, wu_ref, b0_ref, b1_ref,
             bu_ref, out_ref):
    f32 = jnp.float32
    e = emb_ref[...]
    d = e.shape[1]
    wub = wu_ref[d:]
    a0 = wub + jnp.dot(w0_ref[...], wub, preferred_element_type=f32)
    a1 = wub + jnp.dot(w1_ref[...], wub, preferred_element_type=f32)
    y0 = jnp.dot(e, a0, preferred_element_type=f32) \
        + jnp.dot(b0_ref[...], wub, preferred_element_type=f32)
    y1 = jnp.dot(e, a1, preferred_element_type=f32) \
        + jnp.dot(b1_ref[...], wub, preferred_element_type=f32)
    cnt = cnt_ref[...]
    c0 = jnp.sum(cnt[:_NW], axis=0)[:, None]
    c1 = jnp.sum(cnt[_NW:], axis=0)[:, None]
    h = (jnp.dot(e, wu_ref[:d], preferred_element_type=f32)
         + c0 * y0 + c1 * y1 + bu_ref[...])
    out_ref[...] = jnp.maximum(h, 0.0)


def kernel(node_embeddings, rel0_indices, rel1_indices,
           W_msg_0, b_msg_0, W_msg_1, b_msg_1, W_upd, b_upd):
    n, d = node_embeddings.shape
    idx0 = rel0_indices.astype(jnp.int32)
    idx1 = rel1_indices.astype(jnp.int32)

    counts = _sc_histogram(idx0, idx1).reshape(2 * _NW, _NPAD)

    blk = 2560
    grid = (n + blk - 1) // blk
    return pl.pallas_call(
        _tc_body,
        grid=(grid,),
        in_specs=[
            pl.BlockSpec((2 * _NW, blk), lambda i: (0, i)),
            pl.BlockSpec((blk, d), lambda i: (i, 0)),
            pl.BlockSpec((d, d), lambda i: (0, 0)),
            pl.BlockSpec((d, d), lambda i: (0, 0)),
            pl.BlockSpec((2 * d, d), lambda i: (0, 0)),
            pl.BlockSpec((1, d), lambda i: (0, 0)),
            pl.BlockSpec((1, d), lambda i: (0, 0)),
            pl.BlockSpec((1, d), lambda i: (0, 0)),
        ],
        out_specs=pl.BlockSpec((blk, d), lambda i: (i, 0)),
        out_shape=jax.ShapeDtypeStruct((n, d), jnp.float32),
    )(counts, node_embeddings, W_msg_0, W_msg_1, W_upd,
      b_msg_0.reshape(1, d), b_msg_1.reshape(1, d), b_upd.reshape(1, d))
